# baseline (speedup 1.0000x reference)
import functools

import jax
import jax.numpy as jnp
from jax import lax
from jax.experimental import pallas as pl
from jax.experimental.pallas import tpu as pltpu
from jax.experimental.pallas import tpu_sc as plsc

N = 50000
E = 800000
DIM = 64
DS = 32
NB = 8
ZDIM = 16

NP = 50176
BN = 512

E2 = 819200
CH = 256
NCH = E2 // CH
GRP = 4
BE = 2048

NC = 2
NS = 16
G_ROWS = NCH // (NC * NS)
G_GRPS = G_ROWS // GRP
S_ROWS = NCH // NS
SGRP = 1
S_NGRP = S_ROWS // SGRP
ACC_ROWS = NP + 16
ROWS_PER = NP // NS
ZR = 196


def _silu(x):
    return x / (1.0 + jnp.exp(-x))


def _tssr3(x):
    return x * jnp.exp(jnp.log1p(x * x) * (-1.0 / 3.0))


def _node0_body(sp_ref, wz_ref, ws_ref, bs_ref, xi_ref, si_ref):
    iot = lax.broadcasted_iota(jnp.int32, (BN, ZDIM), 1)
    oh = (sp_ref[...] == iot).astype(jnp.float32)
    xi = jnp.dot(oh, wz_ref[...], preferred_element_type=jnp.float32)
    si = _silu(jnp.dot(xi, ws_ref[...], preferred_element_type=jnp.float32)
               + bs_ref[...])
    xi_ref[...] = xi
    si_ref[...] = si


def _node0(species2d, W_z, W_src, b_src):
    return pl.pallas_call(
        _node0_body,
        grid=(NP // BN,),
        in_specs=[
            pl.BlockSpec((BN, 1), lambda i: (i, 0)),
            pl.BlockSpec((ZDIM, DIM), lambda i: (0, 0)),
            pl.BlockSpec((DIM, DS), lambda i: (0, 0)),
            pl.BlockSpec((1, DS), lambda i: (0, 0)),
        ],
        out_specs=[
            pl.BlockSpec((BN, DIM), lambda i: (i, 0)),
            pl.BlockSpec((BN, DS), lambda i: (i, 0)),
        ],
        out_shape=[
            jax.ShapeDtypeStruct((NP, DIM), jnp.float32),
            jax.ShapeDtypeStruct((NP, DS), jnp.float32),
        ],
    )(species2d, W_z, W_src, b_src)


def _msg_body(s_ref, d_ref, wm_ref, lo_ref, hi_ref):
    s = s_ref[...]
    d = d_ref[...]
    K = NB * DS
    dd = jnp.dot(d, jnp.ones((1, K), jnp.float32),
                 preferred_element_type=jnp.float32)
    col = lax.broadcasted_iota(jnp.int32, (DS, K), 1)
    row = lax.broadcasted_iota(jnp.int32, (DS, K), 0)
    tile_mat = (col % DS == row).astype(jnp.float32)
    s8 = jnp.dot(s, tile_mat, preferred_element_type=jnp.float32)
    cc = ((lax.broadcasted_iota(jnp.int32, (1, K), 1) // DS)
          .astype(jnp.float32) * (1.0 / (NB - 1.0)))
    db = dd - cc
    p = jnp.exp(-16.0 * db * db) * s8
    acc = jnp.dot(p, wm_ref[...], preferred_element_type=jnp.float32)
    lo_ref[...] = acc[:, :DS]
    hi_ref[...] = acc[:, DS:]


def _msg(sij, dist2d, W_m):
    return pl.pallas_call(
        _msg_body,
        grid=(E2 // BE,),
        in_specs=[
            pl.BlockSpec((BE, DS), lambda i: (i, 0)),
            pl.BlockSpec((BE, 1), lambda i: (i, 0)),
            pl.BlockSpec((NB * DS, DIM), lambda i: (0, 0)),
        ],
        out_specs=[
            pl.BlockSpec((BE, DS), lambda i: (i, 0)),
            pl.BlockSpec((BE, DS), lambda i: (i, 0)),
        ],
        out_shape=[
            jax.ShapeDtypeStruct((E2, DS), jnp.float32),
            jax.ShapeDtypeStruct((E2, DS), jnp.float32),
        ],
    )(sij, dist2d, W_m)


def _update_body(with_src, xi_ref, alo_ref, ahi_ref, wmx_ref, bm_ref,
                 ws_ref, bs_ref, y_ref, si_ref):
    xi = xi_ref[...]
    pre = jnp.dot(xi, wmx_ref[...], preferred_element_type=jnp.float32)
    pre = pre + bm_ref[...]
    pre = pre + jnp.concatenate([alo_ref[0], ahi_ref[0]], axis=1)
    dx = _silu(pre)
    y = xi + _tssr3(dx)
    y_ref[...] = y
    if with_src:
        si_ref[...] = _silu(
            jnp.dot(y, ws_ref[...], preferred_element_type=jnp.float32)
            + bs_ref[...])


def _update(xi, acc2, W_mx, b_mix, W_src, b_src, with_src):
    out_specs = [pl.BlockSpec((BN, DIM), lambda i: (i, 0))]
    out_shape = [jax.ShapeDtypeStruct((NP, DIM), jnp.float32)]
    if with_src:
        out_specs.append(pl.BlockSpec((BN, DS), lambda i: (i, 0)))
        out_shape.append(jax.ShapeDtypeStruct((NP, DS), jnp.float32))

    def body(xi_ref, alo_ref, ahi_ref, wmx_ref, bm_ref, ws_ref, bs_ref,
             y_ref, si_ref=None):
        _update_body(with_src, xi_ref, alo_ref, ahi_ref, wmx_ref, bm_ref,
                     ws_ref, bs_ref, y_ref, si_ref)

    return pl.pallas_call(
        body,
        grid=(NP // BN,),
        in_specs=[
            pl.BlockSpec((BN, DIM), lambda i: (i, 0)),
            pl.BlockSpec((1, BN, DS), lambda i: (0, i, 0)),
            pl.BlockSpec((1, BN, DS), lambda i: (1, i, 0)),
            pl.BlockSpec((DIM, DIM), lambda i: (0, 0)),
            pl.BlockSpec((1, DIM), lambda i: (0, 0)),
            pl.BlockSpec((DIM, DS), lambda i: (0, 0)),
            pl.BlockSpec((1, DS), lambda i: (0, 0)),
        ],
        out_specs=out_specs,
        out_shape=out_shape,
    )(xi, acc2, acc2, W_mx, b_mix, W_src, b_src)


def _make_gather():
    mesh = plsc.VectorSubcoreMesh(core_axis_name="c", subcore_axis_name="s",
                                  num_cores=NC, num_subcores=NS)

    @functools.partial(
        pl.kernel,
        out_type=jax.ShapeDtypeStruct((NCH, CH, DS), jnp.float32),
        mesh=mesh,
        compiler_params=pltpu.CompilerParams(use_tc_tiling_on_sc=False),
        scratch_types=[
            pltpu.VMEM((G_ROWS, CH), jnp.int32),
            pltpu.VMEM((GRP, CH, DS), jnp.float32),
            pltpu.SemaphoreType.DMA,
        ],
    )
    def gath(table_hbm, idx_hbm, out_hbm, idx_all, rows_v, sem):
        wid = lax.axis_index("s") * NC + lax.axis_index("c")
        row0 = wid * G_ROWS
        pltpu.sync_copy(idx_hbm.at[pl.ds(row0, G_ROWS)], idx_all)

        def group(g, carry):
            descs = []
            for k in range(GRP):
                descs.append(pltpu.async_copy(
                    table_hbm.at[idx_all.at[g * GRP + k]],
                    rows_v.at[k], sem))
            for d in descs:
                d.wait()
            pltpu.sync_copy(rows_v, out_hbm.at[pl.ds(row0 + g * GRP, GRP)])
            return carry

        lax.fori_loop(0, G_GRPS, group, 0)

    return gath


_gather_cache = []


def _gather(table, idx):
    if not _gather_cache:
        _gather_cache.append(_make_gather())
    return _gather_cache[0](table, idx)


def _make_scatter():
    mesh = plsc.VectorSubcoreMesh(core_axis_name="c", subcore_axis_name="s",
                                  num_cores=NC, num_subcores=NS)

    @functools.partial(
        pl.kernel,
        out_type=jax.ShapeDtypeStruct((NC, NP, DS), jnp.float32),
        mesh=mesh,
        compiler_params=pltpu.CompilerParams(use_tc_tiling_on_sc=False),
        scratch_types=[
            pltpu.VMEM((SGRP, CH), jnp.int32),
            pltpu.VMEM((SGRP, CH), jnp.int32),
            pltpu.VMEM((SGRP, CH, DS), jnp.float32),
            pltpu.VMEM((SGRP, CH, DS), jnp.float32),
            pltpu.VMEM((ZR, DS), jnp.float32),
            pltpu.VMEM_SHARED((ACC_ROWS, DS), jnp.float32),
            pltpu.SemaphoreType.DMA,
            pltpu.SemaphoreType.DMA,
            pltpu.SemaphoreType.DMA,
        ],
    )
    def scat(lo_hbm, hi_hbm, dst_hbm, out_hbm, idx0, idx1, mb0, mb1,
             zbuf, acc_sh, sem_i, sem_m, sem_s):
        c = lax.axis_index("c")
        s = lax.axis_index("s")
        idx_b = (idx0, idx1)
        mb_b = (mb0, mb1)

        def zrow(i, carry):
            zbuf[i, 0:16] = jnp.zeros((16,), jnp.float32)
            zbuf[i, 16:32] = jnp.zeros((16,), jnp.float32)
            return carry

        lax.fori_loop(0, ZR, zrow, 0)

        def zcopy(k, carry):
            pltpu.sync_copy(zbuf,
                            acc_sh.at[pl.ds(s * ROWS_PER + k * ZR, ZR)])
            return carry

        lax.fori_loop(0, ROWS_PER // ZR, zcopy, 0)

        @pl.when(s == 0)
        def _():
            pltpu.sync_copy(zbuf.at[pl.ds(0, 16)],
                            acc_sh.at[pl.ds(NP, 16)])

        plsc.subcore_barrier()

        row0 = s * S_ROWS

        def process(msg_hbm):
            def start_load(g, b):
                rows = pl.ds(row0 + g * SGRP, SGRP)
                pltpu.async_copy(dst_hbm.at[rows], idx_b[b], sem_i)
                pltpu.async_copy(msg_hbm.at[rows], mb_b[b], sem_m)

            def wait_load(g, b):
                rows = pl.ds(row0 + g * SGRP, SGRP)
                pltpu.make_async_copy(dst_hbm.at[rows], idx_b[b],
                                      sem_i).wait()
                pltpu.make_async_copy(msg_hbm.at[rows], mb_b[b],
                                      sem_m).wait()

            def run_group(g, b):
                wait_load(g, b)

                @pl.when(g + 1 < S_NGRP)
                def _():
                    start_load(g + 1, 1 - b)

                iv = idx_b[b]
                descs = []
                for k in range(SGRP):
                    descs.append(pltpu.async_copy(
                        mb_b[b].at[k], acc_sh.at[iv.at[k]],
                        sem_s, add=True))
                for d in descs:
                    d.wait()

            start_load(0, 0)

            def pair(j, carry):
                run_group(2 * j, 0)
                run_group(2 * j + 1, 1)
                return carry

            lax.fori_loop(0, S_NGRP // 2, pair, 0)

        @pl.when(c == 0)
        def _():
            process(lo_hbm)

        @pl.when(c == 1)
        def _():
            process(hi_hbm)

        plsc.subcore_barrier()
        pltpu.sync_copy(acc_sh.at[pl.ds(s * ROWS_PER, ROWS_PER)],
                        out_hbm.at[c, pl.ds(s * ROWS_PER, ROWS_PER)])

    return scat


_scatter_cache = []


def _scatter(lo, hi, dst):
    if not _scatter_cache:
        _scatter_cache.append(_make_scatter())
    return _scatter_cache[0](lo, hi, dst)


def kernel(species, edge_src, edge_dst, distances, W_z, W_src0, b_src0,
           W_mix0, b_mix0, W_src1, b_src1, W_mix1, b_mix1):
    species2d = jnp.pad(species.astype(jnp.int32), (0, NP - N)).reshape(NP, 1)
    dist2d = jnp.pad(distances, (0, E2 - E)).reshape(E2, 1)
    esrc2 = jnp.pad(edge_src.astype(jnp.int32), (0, E2 - E)).reshape(NCH, CH)
    edst2 = jnp.pad(edge_dst.astype(jnp.int32), (0, E2 - E),
                    constant_values=NP).reshape(NCH, CH)

    xi, si = _node0(species2d, W_z, W_src0, b_src0.reshape(1, DS))

    for l in range(2):
        W_mix, b_mix = (W_mix0, b_mix0) if l == 0 else (W_mix1, b_mix1)
        sij = _gather(si, esrc2).reshape(E2, DS)
        lo, hi = _msg(sij, dist2d, W_mix[DIM:])
        acc2 = _scatter(lo.reshape(NCH, CH, DS), hi.reshape(NCH, CH, DS),
                        edst2)
        if l == 0:
            xi, si = _update(xi, acc2, W_mix[:DIM], b_mix.reshape(1, DIM),
                             W_src1, b_src1.reshape(1, DS), True)
        else:
            (xi,) = _update(xi, acc2, W_mix[:DIM], b_mix.reshape(1, DIM),
                            W_src1, b_src1.reshape(1, DS), False)
    return xi[:N]


# speedup vs baseline: 1.0099x; 1.0099x over previous
import functools

import jax
import jax.numpy as jnp
from jax import lax
from jax.experimental import pallas as pl
from jax.experimental.pallas import tpu as pltpu
from jax.experimental.pallas import tpu_sc as plsc

N = 50000
E = 800000
DIM = 64
DS = 32
NB = 8
ZDIM = 16

NP = 50176
BN = 512

E2 = 819200
CH = 256
NCH = E2 // CH
GRP = 4
BE = 2048

NC = 2
NS = 16
G_ROWS = NCH // (NC * NS)
G_GRPS = G_ROWS // GRP
S_ROWS = NCH // NS
SGRP = 1
S_NGRP = S_ROWS // SGRP
ACC_ROWS = NP + 16
ROWS_PER = NP // NS
ZR = 196


def _silu(x):
    return x / (1.0 + jnp.exp(-x))


def _tssr3(x):
    return x * jnp.exp(jnp.log1p(x * x) * (-1.0 / 3.0))


def _node0_body(sp_ref, wz_ref, ws_ref, bs_ref, xi_ref, si_ref):
    iot = lax.broadcasted_iota(jnp.int32, (BN, ZDIM), 1)
    oh = (sp_ref[...] == iot).astype(jnp.float32)
    xi = jnp.dot(oh, wz_ref[...], preferred_element_type=jnp.float32)
    si = _silu(jnp.dot(xi, ws_ref[...], preferred_element_type=jnp.float32)
               + bs_ref[...])
    xi_ref[...] = xi
    si_ref[...] = si


def _node0(species2d, W_z, W_src, b_src):
    return pl.pallas_call(
        _node0_body,
        grid=(NP // BN,),
        in_specs=[
            pl.BlockSpec((BN, 1), lambda i: (i, 0)),
            pl.BlockSpec((ZDIM, DIM), lambda i: (0, 0)),
            pl.BlockSpec((DIM, DS), lambda i: (0, 0)),
            pl.BlockSpec((1, DS), lambda i: (0, 0)),
        ],
        out_specs=[
            pl.BlockSpec((BN, DIM), lambda i: (i, 0)),
            pl.BlockSpec((BN, DS), lambda i: (i, 0)),
        ],
        out_shape=[
            jax.ShapeDtypeStruct((NP, DIM), jnp.float32),
            jax.ShapeDtypeStruct((NP, DS), jnp.float32),
        ],
    )(species2d, W_z, W_src, b_src)


def _msg_body(s_ref, d_ref, wm_ref, lo_ref, hi_ref):
    s = s_ref[...]
    d = d_ref[...]
    K = NB * DS
    dd = jnp.dot(d, jnp.ones((1, K), jnp.float32),
                 preferred_element_type=jnp.float32)
    col = lax.broadcasted_iota(jnp.int32, (DS, K), 1)
    row = lax.broadcasted_iota(jnp.int32, (DS, K), 0)
    tile_mat = (col % DS == row).astype(jnp.float32)
    s8 = jnp.dot(s, tile_mat, preferred_element_type=jnp.float32)
    cc = ((lax.broadcasted_iota(jnp.int32, (1, K), 1) // DS)
          .astype(jnp.float32) * (1.0 / (NB - 1.0)))
    db = dd - cc
    p = jnp.exp(-16.0 * db * db) * s8
    acc = jnp.dot(p, wm_ref[...], preferred_element_type=jnp.float32)
    lo_ref[...] = acc[:, :DS]
    hi_ref[...] = acc[:, DS:]


def _msg(sij, dist2d, W_m):
    return pl.pallas_call(
        _msg_body,
        grid=(E2 // BE,),
        in_specs=[
            pl.BlockSpec((BE, DS), lambda i: (i, 0)),
            pl.BlockSpec((BE, 1), lambda i: (i, 0)),
            pl.BlockSpec((NB * DS, DIM), lambda i: (0, 0)),
        ],
        out_specs=[
            pl.BlockSpec((BE, DS), lambda i: (i, 0)),
            pl.BlockSpec((BE, DS), lambda i: (i, 0)),
        ],
        out_shape=[
            jax.ShapeDtypeStruct((E2, DS), jnp.float32),
            jax.ShapeDtypeStruct((E2, DS), jnp.float32),
        ],
    )(sij, dist2d, W_m)


def _update_body(with_src, xi_ref, alo_ref, ahi_ref, wmx_ref, bm_ref,
                 ws_ref, bs_ref, y_ref, si_ref):
    xi = xi_ref[...]
    pre = jnp.dot(xi, wmx_ref[...], preferred_element_type=jnp.float32)
    pre = pre + bm_ref[...]
    pre = pre + jnp.concatenate([alo_ref[0], ahi_ref[0]], axis=1)
    dx = _silu(pre)
    y = xi + _tssr3(dx)
    y_ref[...] = y
    if with_src:
        si_ref[...] = _silu(
            jnp.dot(y, ws_ref[...], preferred_element_type=jnp.float32)
            + bs_ref[...])


def _update(xi, acc2, W_mx, b_mix, W_src, b_src, with_src):
    out_specs = [pl.BlockSpec((BN, DIM), lambda i: (i, 0))]
    out_shape = [jax.ShapeDtypeStruct((NP, DIM), jnp.float32)]
    if with_src:
        out_specs.append(pl.BlockSpec((BN, DS), lambda i: (i, 0)))
        out_shape.append(jax.ShapeDtypeStruct((NP, DS), jnp.float32))

    def body(xi_ref, alo_ref, ahi_ref, wmx_ref, bm_ref, ws_ref, bs_ref,
             y_ref, si_ref=None):
        _update_body(with_src, xi_ref, alo_ref, ahi_ref, wmx_ref, bm_ref,
                     ws_ref, bs_ref, y_ref, si_ref)

    return pl.pallas_call(
        body,
        grid=(NP // BN,),
        in_specs=[
            pl.BlockSpec((BN, DIM), lambda i: (i, 0)),
            pl.BlockSpec((1, BN, DS), lambda i: (0, i, 0)),
            pl.BlockSpec((1, BN, DS), lambda i: (1, i, 0)),
            pl.BlockSpec((DIM, DIM), lambda i: (0, 0)),
            pl.BlockSpec((1, DIM), lambda i: (0, 0)),
            pl.BlockSpec((DIM, DS), lambda i: (0, 0)),
            pl.BlockSpec((1, DS), lambda i: (0, 0)),
        ],
        out_specs=out_specs,
        out_shape=out_shape,
    )(xi, acc2, acc2, W_mx, b_mix, W_src, b_src)


def _make_gather():
    mesh = plsc.VectorSubcoreMesh(core_axis_name="c", subcore_axis_name="s",
                                  num_cores=NC, num_subcores=NS)

    @functools.partial(
        pl.kernel,
        out_type=jax.ShapeDtypeStruct((NCH, CH, DS), jnp.float32),
        mesh=mesh,
        compiler_params=pltpu.CompilerParams(use_tc_tiling_on_sc=False),
        scratch_types=[
            pltpu.VMEM((G_ROWS, CH), jnp.int32),
            pltpu.VMEM((GRP, CH, DS), jnp.float32),
            pltpu.VMEM((GRP, CH, DS), jnp.float32),
            pltpu.SemaphoreType.DMA,
            pltpu.SemaphoreType.DMA,
            pltpu.SemaphoreType.DMA,
        ],
    )
    def gath(table_hbm, idx_hbm, out_hbm, idx_all, rv0, rv1, sg0, sg1,
             sem_o):
        wid = lax.axis_index("s") * NC + lax.axis_index("c")
        row0 = wid * G_ROWS
        pltpu.sync_copy(idx_hbm.at[pl.ds(row0, G_ROWS)], idx_all)
        rv = (rv0, rv1)
        sg = (sg0, sg1)

        def fire(g, b):
            for k in range(GRP):
                pltpu.async_copy(table_hbm.at[idx_all.at[g * GRP + k]],
                                 rv[b].at[k], sg[b])

        def wait_g(g, b):
            for k in range(GRP):
                pltpu.make_async_copy(
                    table_hbm.at[idx_all.at[g * GRP + k]],
                    rv[b].at[k], sg[b]).wait()

        def wb_fire(g, b):
            pltpu.async_copy(rv[b], out_hbm.at[pl.ds(row0 + g * GRP, GRP)],
                             sem_o)

        def wb_wait(g, b):
            pltpu.make_async_copy(
                rv[b], out_hbm.at[pl.ds(row0 + g * GRP, GRP)],
                sem_o).wait()

        fire(0, 0)

        def step(g, b):
            @pl.when(g < G_GRPS)
            def _():
                @pl.when(g >= 1)
                def _():
                    wb_wait(g - 1, 1 - b)

                @pl.when(g + 1 < G_GRPS)
                def _():
                    fire(g + 1, 1 - b)

                wait_g(g, b)
                wb_fire(g, b)

        def pairs(j, carry):
            step(2 * j, 0)
            step(2 * j + 1, 1)
            return carry

        lax.fori_loop(0, (G_GRPS + 2) // 2, pairs, 0)
        wb_wait(G_GRPS - 1, (G_GRPS - 1) % 2)

    return gath


_gather_cache = []


def _gather(table, idx):
    if not _gather_cache:
        _gather_cache.append(_make_gather())
    return _gather_cache[0](table, idx)


def _make_scatter():
    mesh = plsc.VectorSubcoreMesh(core_axis_name="c", subcore_axis_name="s",
                                  num_cores=NC, num_subcores=NS)

    @functools.partial(
        pl.kernel,
        out_type=jax.ShapeDtypeStruct((NC, NP, DS), jnp.float32),
        mesh=mesh,
        compiler_params=pltpu.CompilerParams(use_tc_tiling_on_sc=False),
        scratch_types=[
            pltpu.VMEM((SGRP, CH), jnp.int32),
            pltpu.VMEM((SGRP, CH), jnp.int32),
            pltpu.VMEM((SGRP, CH, DS), jnp.float32),
            pltpu.VMEM((SGRP, CH, DS), jnp.float32),
            pltpu.VMEM((ZR, DS), jnp.float32),
            pltpu.VMEM_SHARED((ACC_ROWS, DS), jnp.float32),
            pltpu.SemaphoreType.DMA,
            pltpu.SemaphoreType.DMA,
            pltpu.SemaphoreType.DMA,
        ],
    )
    def scat(lo_hbm, hi_hbm, dst_hbm, out_hbm, idx0, idx1, mb0, mb1,
             zbuf, acc_sh, sem_i, sem_m, sem_s):
        c = lax.axis_index("c")
        s = lax.axis_index("s")
        idx_b = (idx0, idx1)
        mb_b = (mb0, mb1)

        def zrow(i, carry):
            zbuf[i, 0:16] = jnp.zeros((16,), jnp.float32)
            zbuf[i, 16:32] = jnp.zeros((16,), jnp.float32)
            return carry

        lax.fori_loop(0, ZR, zrow, 0)

        def zcopy(k, carry):
            pltpu.sync_copy(zbuf,
                            acc_sh.at[pl.ds(s * ROWS_PER + k * ZR, ZR)])
            return carry

        lax.fori_loop(0, ROWS_PER // ZR, zcopy, 0)

        @pl.when(s == 0)
        def _():
            pltpu.sync_copy(zbuf.at[pl.ds(0, 16)],
                            acc_sh.at[pl.ds(NP, 16)])

        plsc.subcore_barrier()

        row0 = s * S_ROWS

        def process(msg_hbm):
            def start_load(g, b):
                rows = pl.ds(row0 + g * SGRP, SGRP)
                pltpu.async_copy(dst_hbm.at[rows], idx_b[b], sem_i)
                pltpu.async_copy(msg_hbm.at[rows], mb_b[b], sem_m)

            def wait_load(g, b):
                rows = pl.ds(row0 + g * SGRP, SGRP)
                pltpu.make_async_copy(dst_hbm.at[rows], idx_b[b],
                                      sem_i).wait()
                pltpu.make_async_copy(msg_hbm.at[rows], mb_b[b],
                                      sem_m).wait()

            def run_group(g, b):
                wait_load(g, b)

                @pl.when(g + 1 < S_NGRP)
                def _():
                    start_load(g + 1, 1 - b)

                iv = idx_b[b]
                descs = []
                for k in range(SGRP):
                    descs.append(pltpu.async_copy(
                        mb_b[b].at[k], acc_sh.at[iv.at[k]],
                        sem_s, add=True))
                for d in descs:
                    d.wait()

            start_load(0, 0)

            def pair(j, carry):
                run_group(2 * j, 0)
                run_group(2 * j + 1, 1)
                return carry

            lax.fori_loop(0, S_NGRP // 2, pair, 0)

        @pl.when(c == 0)
        def _():
            process(lo_hbm)

        @pl.when(c == 1)
        def _():
            process(hi_hbm)

        plsc.subcore_barrier()
        pltpu.sync_copy(acc_sh.at[pl.ds(s * ROWS_PER, ROWS_PER)],
                        out_hbm.at[c, pl.ds(s * ROWS_PER, ROWS_PER)])

    return scat


_scatter_cache = []


def _scatter(lo, hi, dst):
    if not _scatter_cache:
        _scatter_cache.append(_make_scatter())
    return _scatter_cache[0](lo, hi, dst)


def kernel(species, edge_src, edge_dst, distances, W_z, W_src0, b_src0,
           W_mix0, b_mix0, W_src1, b_src1, W_mix1, b_mix1):
    species2d = jnp.pad(species.astype(jnp.int32), (0, NP - N)).reshape(NP, 1)
    dist2d = jnp.pad(distances, (0, E2 - E)).reshape(E2, 1)
    esrc2 = jnp.pad(edge_src.astype(jnp.int32), (0, E2 - E)).reshape(NCH, CH)
    edst2 = jnp.pad(edge_dst.astype(jnp.int32), (0, E2 - E),
                    constant_values=NP).reshape(NCH, CH)

    xi, si = _node0(species2d, W_z, W_src0, b_src0.reshape(1, DS))

    for l in range(2):
        W_mix, b_mix = (W_mix0, b_mix0) if l == 0 else (W_mix1, b_mix1)
        sij = _gather(si, esrc2).reshape(E2, DS)
        lo, hi = _msg(sij, dist2d, W_mix[DIM:])
        acc2 = _scatter(lo.reshape(NCH, CH, DS), hi.reshape(NCH, CH, DS),
                        edst2)
        if l == 0:
            xi, si = _update(xi, acc2, W_mix[:DIM], b_mix.reshape(1, DIM),
                             W_src1, b_src1.reshape(1, DS), True)
        else:
            (xi,) = _update(xi, acc2, W_mix[:DIM], b_mix.reshape(1, DIM),
                            W_src1, b_src1.reshape(1, DS), False)
    return xi[:N]
